# Initial kernel scaffold; baseline (speedup 1.0000x reference)
import jax
import jax.numpy as jnp
from jax.experimental import pallas as pl


def kernel(node_features, edge_index, graph_index, params):
    raise NotImplementedError("write your pallas kernel here")


# speedup vs baseline: 4.4578x; 4.4578x over previous
import functools

import jax
import jax.numpy as jnp
from jax import lax
from jax.experimental import pallas as pl
from jax.experimental.pallas import tpu as pltpu
from jax.experimental.pallas import tpu_sc as plsc

N = 10000
E = 320000
NG = 64
DIM = 64
BN_EPS = 1e-5

NC = 2
NS = 16
NW = NC * NS
EDGES_PER_W = E // NW
CHUNK = 80
NCHUNKS = EDGES_PER_W // CHUNK
NP = 10240
ROWS_PER_S = NP // NS

_HIGHEST = lax.Precision.HIGHEST


def _make_seg_sum(D):
  mesh = plsc.VectorSubcoreMesh(core_axis_name="c", subcore_axis_name="s")

  @functools.partial(
      pl.kernel,
      mesh=mesh,
      out_type=jax.ShapeDtypeStruct((NC * NP, D), jnp.float32),
      scratch_types=[
          pltpu.VMEM((CHUNK,), jnp.int32),
          pltpu.VMEM((CHUNK,), jnp.int32),
          pltpu.VMEM((CHUNK, D), jnp.float32),
          pltpu.VMEM_SHARED((NP, D), jnp.float32),
          pltpu.SemaphoreType.DMA,
      ],
      compiler_params=pltpu.CompilerParams(use_tc_tiling_on_sc=False),
      name=f"gin_seg_sum_d{D}",
  )
  def seg_sum(x_hbm, src_hbm, dst_hbm, zeros_hbm, out_hbm,
              idx_s, idx_d, rows, acc, sem):
    c = lax.axis_index("c")
    s = lax.axis_index("s")
    wid = c * NS + s

    r0 = pl.multiple_of(s * ROWS_PER_S, 8)
    pltpu.sync_copy(zeros_hbm.at[pl.ds(r0, ROWS_PER_S)],
                    acc.at[pl.ds(r0, ROWS_PER_S)])
    plsc.subcore_barrier()

    def body(j, carry):
      base = pl.multiple_of(wid * EDGES_PER_W + j * CHUNK, 8)
      pltpu.sync_copy(src_hbm.at[pl.ds(base, CHUNK)], idx_s)
      pltpu.sync_copy(dst_hbm.at[pl.ds(base, CHUNK)], idx_d)
      pltpu.async_copy(x_hbm.at[idx_s], rows, sem).wait()
      pltpu.sync_copy(rows, acc.at[idx_d], add=True)
      return carry

    lax.fori_loop(0, NCHUNKS, body, 0)
    plsc.subcore_barrier()

    out0 = pl.multiple_of(c * NP + s * ROWS_PER_S, 8)
    pltpu.sync_copy(acc.at[pl.ds(r0, ROWS_PER_S)],
                    out_hbm.at[pl.ds(out0, ROWS_PER_S)])

  return seg_sum


_seg_sum_cache = {}


def _seg_sum(D):
  if D not in _seg_sum_cache:
    _seg_sum_cache[D] = _make_seg_sum(D)
  return _seg_sum_cache[D]


def _tc_layer_body(x_ref, agg_ref, gi_ref, w1_ref, b1_ref, w2_ref, b2_ref,
                   gam_ref, bet_ref, y_ref, pool_ref):
  h = x_ref[...] + agg_ref[0] + agg_ref[1]
  h = jnp.maximum(
      jnp.dot(h, w1_ref[...], precision=None,
              preferred_element_type=jnp.float32) + b1_ref[...], 0.0)
  h = jnp.dot(h, w2_ref[...], precision=None,
              preferred_element_type=jnp.float32) + b2_ref[...]
  h = jnp.maximum(h, 0.0)
  mean = jnp.mean(h, axis=0, keepdims=True)
  var = jnp.mean((h - mean) ** 2, axis=0, keepdims=True)
  v = var + BN_EPS
  inv = lax.rsqrt(v)
  inv = inv * (1.5 - 0.5 * v * inv * inv)
  inv = inv * (1.5 - 0.5 * v * inv * inv)
  y = (h - mean) * (inv * gam_ref[...]) + bet_ref[...]
  y_ref[...] = y
  gids = lax.broadcasted_iota(jnp.int32, (NG, 1), 0)
  mask = (gi_ref[...] == gids).astype(jnp.float32)
  pool_ref[...] = jnp.dot(mask, y, precision=None,
                          preferred_element_type=jnp.float32)


def _tc_layer(x, agg2, gi2d, p):
  return pl.pallas_call(
      _tc_layer_body,
      out_shape=[
          jax.ShapeDtypeStruct((N, DIM), jnp.float32),
          jax.ShapeDtypeStruct((NG, DIM), jnp.float32),
      ],
  )(x, agg2, gi2d, p["W1"], p["b1"].reshape(1, DIM), p["W2"],
    p["b2"].reshape(1, DIM), p["gamma"].reshape(1, DIM),
    p["beta"].reshape(1, DIM))


def kernel(node_features, edge_index, graph_index, params):
  src = edge_index[0]
  dst = edge_index[1]
  gi2d = graph_index.reshape(1, N)
  x = node_features
  xs = []
  pools = []
  zeros = {d: jnp.zeros((NP, d), jnp.float32) for d in (node_features.shape[1], DIM)}
  for i in range(5):
    p = params[f"layer_{i}"]
    d = x.shape[1]
    agg2 = _seg_sum(d)(x, src, dst, zeros[d]).reshape(2, NP, d)[:, :N, :]
    y, pooled = _tc_layer(x, agg2, gi2d, p)
    xs.append(y)
    pools.append(pooled)
    x = y
  return jnp.concatenate(pools, axis=1), jnp.concatenate(xs, axis=1)


# speedup vs baseline: 12.3445x; 2.7692x over previous
import functools

import jax
import jax.numpy as jnp
from jax import lax
from jax.experimental import pallas as pl
from jax.experimental.pallas import tpu as pltpu
from jax.experimental.pallas import tpu_sc as plsc

N = 10000
E = 320000
NG = 64
DIM = 64
BN_EPS = 1e-5

NC = 2
NS = 16
NW = NC * NS
EDGES_PER_W = E // NW
NBLK = 5
NBUF = 5
NP = 10240
ROWS_PER_S = NP // NS


def _make_seg_sum(D, chunk):
  nchunks = EDGES_PER_W // chunk
  iblk = nchunks // NBLK
  assert iblk % NBUF == 0 and chunk % 8 == 0 and chunk <= 128
  mesh = plsc.VectorSubcoreMesh(core_axis_name="c", subcore_axis_name="s")
  scratch = ([
      pltpu.VMEM((iblk, chunk), jnp.int32),
      pltpu.VMEM((iblk, chunk), jnp.int32),
      pltpu.VMEM((iblk, chunk), jnp.int32),
      pltpu.VMEM((iblk, chunk), jnp.int32),
      pltpu.SemaphoreType.DMA,
      pltpu.SemaphoreType.DMA,
  ] + [pltpu.VMEM((chunk, D), jnp.float32) for _ in range(NBUF)]
    + [pltpu.SemaphoreType.DMA for _ in range(NBUF)]
    + [pltpu.VMEM_SHARED((NP, D), jnp.float32)])

  @functools.partial(
      pl.kernel,
      mesh=mesh,
      out_type=jax.ShapeDtypeStruct((NC * NP, D), jnp.float32),
      scratch_types=scratch,
      compiler_params=pltpu.CompilerParams(use_tc_tiling_on_sc=False),
      name=f"gin_seg_sum_d{D}",
  )
  def seg_sum(x_hbm, src_hbm, dst_hbm, zeros_hbm, out_hbm,
              is0, id0, is1, id1, semi0, semi1, *rest):
    rows = rest[:NBUF]
    sems = rest[NBUF:2 * NBUF]
    acc = rest[2 * NBUF]
    idx = [(is0, id0, semi0), (is1, id1, semi1)]
    c = lax.axis_index("c")
    s = lax.axis_index("s")
    wid = c * NS + s

    pltpu.sync_copy(src_hbm.at[wid, 0], is0)
    pltpu.sync_copy(dst_hbm.at[wid, 0], id0)

    r0 = pl.multiple_of(s * ROWS_PER_S, 8)
    pltpu.sync_copy(zeros_hbm.at[pl.ds(r0, ROWS_PER_S)],
                    acc.at[pl.ds(r0, ROWS_PER_S)])
    plsc.subcore_barrier()

    for blk in range(NBLK):
      isv, idv, _ = idx[blk % 2]
      if blk + 1 < NBLK:
        nsv, ndv, nsem = idx[(blk + 1) % 2]
        pltpu.async_copy(src_hbm.at[wid, blk + 1], nsv, nsem)
        pltpu.async_copy(dst_hbm.at[wid, blk + 1], ndv, nsem)

      for b in range(NBUF):
        pltpu.async_copy(x_hbm.at[isv.at[b]], rows[b], sems[b])

      def inner(it, carry):
        jbase = it * NBUF
        for b in range(NBUF):
          jj = jbase + b
          pltpu.make_async_copy(x_hbm.at[isv.at[jj]], rows[b],
                                sems[b]).wait()
          pltpu.sync_copy(rows[b], acc.at[idv.at[jj]], add=True)

          @pl.when(jj + NBUF < iblk)
          def _():
            pltpu.async_copy(x_hbm.at[isv.at[jj + NBUF]], rows[b], sems[b])
        return carry

      lax.fori_loop(0, iblk // NBUF, inner, 0, unroll=False)

      if blk + 1 < NBLK:
        nsv, ndv, nsem = idx[(blk + 1) % 2]
        pltpu.make_async_copy(src_hbm.at[wid, blk + 1], nsv, nsem).wait()
        pltpu.make_async_copy(dst_hbm.at[wid, blk + 1], ndv, nsem).wait()

    plsc.subcore_barrier()
    out0 = pl.multiple_of(c * NP + s * ROWS_PER_S, 8)
    pltpu.sync_copy(acc.at[pl.ds(r0, ROWS_PER_S)],
                    out_hbm.at[pl.ds(out0, ROWS_PER_S)])

  return seg_sum


_seg_sum_cache = {}


def _seg_sum(D):
  if D not in _seg_sum_cache:
    _seg_sum_cache[D] = _make_seg_sum(D, 40 if D > 64 else 80)
  return _seg_sum_cache[D]


def _tc_layer_body(x_ref, agg_ref, gi_ref, w1_ref, b1_ref, w2_ref, b2_ref,
                   gam_ref, bet_ref, y_ref, pool_ref):
  h = x_ref[...] + (agg_ref[pl.ds(0, N), :] + agg_ref[pl.ds(NP, N), :])
  h = jnp.maximum(
      jnp.dot(h, w1_ref[...],
              preferred_element_type=jnp.float32) + b1_ref[...], 0.0)
  h = jnp.dot(h, w2_ref[...],
              preferred_element_type=jnp.float32) + b2_ref[...]
  h = jnp.maximum(h, 0.0)
  mean = jnp.mean(h, axis=0, keepdims=True)
  var = jnp.mean((h - mean) ** 2, axis=0, keepdims=True)
  v = var + BN_EPS
  inv = lax.rsqrt(v)
  inv = inv * (1.5 - 0.5 * v * inv * inv)
  inv = inv * (1.5 - 0.5 * v * inv * inv)
  y = (h - mean) * (inv * gam_ref[...]) + bet_ref[...]
  y_ref[...] = y
  gids = lax.broadcasted_iota(jnp.int32, (NG, 1), 0)
  mask = (gi_ref[...] == gids).astype(jnp.float32)
  pool_ref[...] = jnp.dot(mask, y, preferred_element_type=jnp.float32)


def _tc_layer(x, agg2, gi2d, p):
  return pl.pallas_call(
      _tc_layer_body,
      out_shape=[
          jax.ShapeDtypeStruct((N, DIM), jnp.float32),
          jax.ShapeDtypeStruct((NG, DIM), jnp.float32),
      ],
  )(x, agg2, gi2d, p["W1"], p["b1"].reshape(1, DIM), p["W2"],
    p["b2"].reshape(1, DIM), p["gamma"].reshape(1, DIM),
    p["beta"].reshape(1, DIM))


def kernel(node_features, edge_index, graph_index, params):
  gi2d = graph_index.reshape(1, N)
  x = node_features
  xs = []
  pools = []
  d0 = node_features.shape[1]
  zeros = {d: jnp.zeros((NP, d), jnp.float32) for d in (d0, DIM)}
  edge4 = {}
  for d in (d0, DIM):
    chunk = 40 if d > 64 else 80
    nchunks = EDGES_PER_W // chunk
    edge4[d] = (edge_index[0].reshape(NW, NBLK, nchunks // NBLK, chunk),
                edge_index[1].reshape(NW, NBLK, nchunks // NBLK, chunk))
  for i in range(5):
    p = params[f"layer_{i}"]
    d = x.shape[1]
    src4, dst4 = edge4[d]
    agg2 = _seg_sum(d)(x, src4, dst4, zeros[d])
    y, pooled = _tc_layer(x, agg2, gi2d, p)
    xs.append(y)
    pools.append(pooled)
    x = y
  return jnp.concatenate(pools, axis=1), jnp.concatenate(xs, axis=1)


# speedup vs baseline: 13.9952x; 1.1337x over previous
import functools

import jax
import jax.numpy as jnp
from jax import lax
from jax.experimental import pallas as pl
from jax.experimental.pallas import tpu as pltpu
from jax.experimental.pallas import tpu_sc as plsc

N = 10000
E = 320000
NG = 64
DIM = 64
BN_EPS = 1e-5

NC = 2
NS = 16
NW = NC * NS
EDGES_PER_W = E // NW
NBLK = 5
NBUF = 5
NP = 10240
ROWS_PER_S = NP // NS


def _make_seg_sum(D, chunk):
  nchunks = EDGES_PER_W // chunk
  iblk = nchunks // NBLK
  assert iblk % NBUF == 0 and chunk % 8 == 0 and chunk <= 128
  mesh = plsc.VectorSubcoreMesh(core_axis_name="c", subcore_axis_name="s")
  scratch = ([
      pltpu.VMEM((iblk, chunk), jnp.int32),
      pltpu.VMEM((iblk, chunk), jnp.int32),
      pltpu.VMEM((iblk, chunk), jnp.int32),
      pltpu.VMEM((iblk, chunk), jnp.int32),
      pltpu.SemaphoreType.DMA,
      pltpu.SemaphoreType.DMA,
  ] + [pltpu.VMEM((chunk, D), jnp.float32) for _ in range(NBUF)]
    + [pltpu.SemaphoreType.DMA for _ in range(NBUF)]
    + [pltpu.VMEM_SHARED((NP, D), jnp.float32)])

  @functools.partial(
      pl.kernel,
      mesh=mesh,
      out_type=jax.ShapeDtypeStruct((NC * NP, D), jnp.float32),
      scratch_types=scratch,
      compiler_params=pltpu.CompilerParams(use_tc_tiling_on_sc=False),
      name=f"gin_seg_sum_d{D}",
  )
  def seg_sum(x_hbm, src_hbm, dst_hbm, zeros_hbm, out_hbm,
              is0, id0, is1, id1, semi0, semi1, *rest):
    rows = rest[:NBUF]
    sems = rest[NBUF:2 * NBUF]
    acc = rest[2 * NBUF]
    idx = [(is0, id0, semi0), (is1, id1, semi1)]
    c = lax.axis_index("c")
    s = lax.axis_index("s")
    wid = c * NS + s

    pltpu.sync_copy(src_hbm.at[wid, 0], is0)
    pltpu.sync_copy(dst_hbm.at[wid, 0], id0)

    r0 = pl.multiple_of(s * ROWS_PER_S, 8)
    pltpu.sync_copy(zeros_hbm.at[pl.ds(r0, ROWS_PER_S)],
                    acc.at[pl.ds(r0, ROWS_PER_S)])
    plsc.subcore_barrier()

    for blk in range(NBLK):
      isv, idv, _ = idx[blk % 2]
      if blk + 1 < NBLK:
        nsv, ndv, nsem = idx[(blk + 1) % 2]
        pltpu.async_copy(src_hbm.at[wid, blk + 1], nsv, nsem)
        pltpu.async_copy(dst_hbm.at[wid, blk + 1], ndv, nsem)

      for b in range(NBUF):
        pltpu.async_copy(x_hbm.at[isv.at[b]], rows[b], sems[b])

      def inner(it, carry):
        jbase = it * NBUF
        for b in range(NBUF):
          jj = jbase + b
          pltpu.make_async_copy(x_hbm.at[isv.at[jj]], rows[b],
                                sems[b]).wait()
          pltpu.sync_copy(rows[b], acc.at[idv.at[jj]], add=True)

          @pl.when(jj + NBUF < iblk)
          def _():
            pltpu.async_copy(x_hbm.at[isv.at[jj + NBUF]], rows[b], sems[b])
        return carry

      lax.fori_loop(0, iblk // NBUF, inner, 0, unroll=False)

      if blk + 1 < NBLK:
        nsv, ndv, nsem = idx[(blk + 1) % 2]
        pltpu.make_async_copy(src_hbm.at[wid, blk + 1], nsv, nsem).wait()
        pltpu.make_async_copy(dst_hbm.at[wid, blk + 1], ndv, nsem).wait()

    plsc.subcore_barrier()
    out0 = pl.multiple_of(c * NP + s * ROWS_PER_S, 8)
    pltpu.sync_copy(acc.at[pl.ds(r0, ROWS_PER_S)],
                    out_hbm.at[pl.ds(out0, ROWS_PER_S)])

  return seg_sum


_seg_sum_cache = {}


def _seg_sum(D):
  if D not in _seg_sum_cache:
    _seg_sum_cache[D] = _make_seg_sum(D, 40 if D > 64 else 80)
  return _seg_sum_cache[D]


def _bn_mlp_pool(x_ref, agg_ref, gi_ref, w1_ref, b1_ref, w2_ref, b2_ref,
                 gam_ref, bet_ref):
  h = x_ref[...] + (agg_ref[pl.ds(0, N), :] + agg_ref[pl.ds(NP, N), :])
  h = jnp.maximum(
      jnp.dot(h, w1_ref[...],
              preferred_element_type=jnp.float32) + b1_ref[...], 0.0)
  h = jnp.dot(h, w2_ref[...],
              preferred_element_type=jnp.float32) + b2_ref[...]
  h = jnp.maximum(h, 0.0)
  mean = jnp.mean(h, axis=0, keepdims=True)
  var = jnp.mean((h - mean) ** 2, axis=0, keepdims=True)
  v = var + BN_EPS
  inv = lax.rsqrt(v)
  inv = inv * (1.5 - 0.5 * v * inv * inv)
  inv = inv * (1.5 - 0.5 * v * inv * inv)
  y = (h - mean) * (inv * gam_ref[...]) + bet_ref[...]
  gids = lax.broadcasted_iota(jnp.int32, (NG, 1), 0)
  mask = (gi_ref[...] == gids).astype(jnp.float32)
  pool = jnp.dot(mask, y, preferred_element_type=jnp.float32)
  return y, pool


def _tc_layer_body(x_ref, agg_ref, gi_ref, w1_ref, b1_ref, w2_ref, b2_ref,
                   gam_ref, bet_ref, y_ref, pool_ref):
  y, pool = _bn_mlp_pool(x_ref, agg_ref, gi_ref, w1_ref, b1_ref, w2_ref,
                         b2_ref, gam_ref, bet_ref)
  y_ref[...] = y
  pool_ref[...] = pool


def _tc_last_body(x_ref, agg_ref, gi_ref, w1_ref, b1_ref, w2_ref, b2_ref,
                  gam_ref, bet_ref, y0, y1, y2, y3, p0, p1, p2, p3,
                  xs_ref, xo_ref):
  y, pool = _bn_mlp_pool(x_ref, agg_ref, gi_ref, w1_ref, b1_ref, w2_ref,
                         b2_ref, gam_ref, bet_ref)
  xs_ref[...] = jnp.concatenate(
      [y0[...], y1[...], y2[...], y3[...], y], axis=1)
  xo_ref[...] = jnp.concatenate(
      [p0[...], p1[...], p2[...], p3[...], pool], axis=1)


def _tc_layer(x, agg2, gi2d, p):
  return pl.pallas_call(
      _tc_layer_body,
      out_shape=[
          jax.ShapeDtypeStruct((N, DIM), jnp.float32),
          jax.ShapeDtypeStruct((NG, DIM), jnp.float32),
      ],
  )(x, agg2, gi2d, p["W1"], p["b1"].reshape(1, DIM), p["W2"],
    p["b2"].reshape(1, DIM), p["gamma"].reshape(1, DIM),
    p["beta"].reshape(1, DIM))


def _tc_last(x, agg2, gi2d, p, ys, pools):
  return pl.pallas_call(
      _tc_last_body,
      out_shape=[
          jax.ShapeDtypeStruct((N, 5 * DIM), jnp.float32),
          jax.ShapeDtypeStruct((NG, 5 * DIM), jnp.float32),
      ],
  )(x, agg2, gi2d, p["W1"], p["b1"].reshape(1, DIM), p["W2"],
    p["b2"].reshape(1, DIM), p["gamma"].reshape(1, DIM),
    p["beta"].reshape(1, DIM), *ys, *pools)


def kernel(node_features, edge_index, graph_index, params):
  gi2d = graph_index.reshape(1, N)
  x = node_features
  xs = []
  pools = []
  d0 = node_features.shape[1]
  zeros = {d: jnp.zeros((NP, d), jnp.float32) for d in (d0, DIM)}
  edge4 = {}
  for d in (d0, DIM):
    chunk = 40 if d > 64 else 80
    nchunks = EDGES_PER_W // chunk
    edge4[d] = (edge_index[0].reshape(NW, NBLK, nchunks // NBLK, chunk),
                edge_index[1].reshape(NW, NBLK, nchunks // NBLK, chunk))
  for i in range(4):
    p = params[f"layer_{i}"]
    d = x.shape[1]
    src4, dst4 = edge4[d]
    agg2 = _seg_sum(d)(x, src4, dst4, zeros[d])
    y, pooled = _tc_layer(x, agg2, gi2d, p)
    xs.append(y)
    pools.append(pooled)
    x = y
  src4, dst4 = edge4[DIM]
  agg2 = _seg_sum(DIM)(x, src4, dst4, zeros[DIM])
  xs_out, x_out = _tc_last(x, agg2, gi2d, params["layer_4"], xs, pools)
  return x_out, xs_out
